# baseline (speedup 1.0000x reference)
import functools

import jax
import jax.numpy as jnp
from jax import lax
from jax.experimental import pallas as pl
from jax.experimental.pallas import tpu as pltpu
from jax.experimental.pallas import tpu_sc as plsc


def kernel(user, item, user_table, item_table):
    B = user.shape[0]
    D = user_table.shape[1]
    info = plsc.get_sparse_core_info()
    NC, NS = info.num_cores, info.num_subcores
    NW = NC * NS
    assert B % (8 * NW) == 0
    b_per_w = B // NW

    mesh = plsc.VectorSubcoreMesh(core_axis_name="c", subcore_axis_name="s")

    CH = b_per_w // 2

    @functools.partial(
        pl.kernel,
        mesh=mesh,
        out_type=(
            jax.ShapeDtypeStruct((B, D), jnp.float32),
            jax.ShapeDtypeStruct((B, D), jnp.float32),
        ),
        scratch_types=[
            pltpu.VMEM((b_per_w,), jnp.int32),
            pltpu.VMEM((b_per_w,), jnp.int32),
            pltpu.VMEM((CH, D), jnp.float32),
            pltpu.VMEM((CH, D), jnp.float32),
            pltpu.SemaphoreType.DMA,
            pltpu.SemaphoreType.DMA,
            pltpu.SemaphoreType.DMA,
            pltpu.SemaphoreType.DMA,
        ],
    )
    def gather2(user_hbm, item_hbm, ut_hbm, it_hbm, uout_hbm, iout_hbm,
                uidx_v, iidx_v, buf_a, buf_b, sga, sgb, swa, swb):
        wid = lax.axis_index("s") * NC + lax.axis_index("c")
        base = wid * b_per_w
        pltpu.sync_copy(user_hbm.at[pl.ds(base, b_per_w)], uidx_v)
        pltpu.sync_copy(item_hbm.at[pl.ds(base, b_per_w)], iidx_v)
        g0 = pltpu.async_copy(ut_hbm.at[uidx_v.at[pl.ds(0, CH)]], buf_a, sga)
        g1 = pltpu.async_copy(ut_hbm.at[uidx_v.at[pl.ds(CH, CH)]], buf_b, sgb)
        g0.wait()
        w0 = pltpu.async_copy(buf_a, uout_hbm.at[pl.ds(base, CH)], swa)
        g1.wait()
        w1 = pltpu.async_copy(buf_b, uout_hbm.at[pl.ds(base + CH, CH)], swb)
        w0.wait()
        g2 = pltpu.async_copy(it_hbm.at[iidx_v.at[pl.ds(0, CH)]], buf_a, sga)
        w1.wait()
        g3 = pltpu.async_copy(it_hbm.at[iidx_v.at[pl.ds(CH, CH)]], buf_b, sgb)
        g2.wait()
        w2 = pltpu.async_copy(buf_a, iout_hbm.at[pl.ds(base, CH)], swa)
        g3.wait()
        w3 = pltpu.async_copy(buf_b, iout_hbm.at[pl.ds(base + CH, CH)], swb)
        w2.wait()
        w3.wait()

    return gather2(user, item, user_table, item_table)
